# baseline (speedup 1.0000x reference)
import jax
import jax.numpy as jnp
from jax.experimental import pallas as pl
from jax.experimental.pallas import tpu as pltpu

B, N, DIN, NH, DH = 256, 128, 128, 8, 16
HID = NH * DH
DOUT = 512

G = 16


def _leaky(x):
    return jnp.maximum(x, 0.2 * x)


def _bf(x):
    return x.astype(jnp.bfloat16)


def _elu(x):
    return jnp.where(x > 0, x, jnp.exp(jnp.minimum(x, 0.0)) - 1.0)


def _att_blocks(blocks):
    p = jnp.concatenate(blocks, axis=0)
    return _bf(p / jnp.sum(p, axis=-1, keepdims=True))


def _gat_kernel(x_ref, adj_ref, W1e_ref, W1d_ref, HM_ref,
                W2e_ref, W2d_ref, W3e_ref, W3d_ref,
                FCW_ref, fcb_ref, o_ref):
    adjs = [adj_ref[g] for g in range(G)]

    x_b = _bf(x_ref[...].reshape(G * N, DIN))
    h1e = jnp.dot(x_b, W1e_ref[...], preferred_element_type=jnp.float32)
    h1_b = _bf(h1e[:, :HID])
    es_all = h1e[:, HID:]
    ed_all = jax.lax.dot_general(
        W1d_ref[...], x_b, (((1,), (1,)), ((), ())),
        preferred_element_type=jnp.float32)

    blocks = []
    for g in range(G):
        for i in range(NH):
            e = _leaky(es_all[g * N:(g + 1) * N, i:i + 1]
                       + ed_all[i:i + 1, g * N:(g + 1) * N])
            blocks.append(jnp.exp2(e) * adjs[g])
    att1 = _att_blocks(blocks)

    hm = HM_ref[...][:, None, :]
    moles = []
    for g in range(G):
        og = jnp.dot(att1[g * NH * N:(g + 1) * NH * N, :],
                     h1_b[g * N:(g + 1) * N, :],
                     preferred_element_type=jnp.float32)
        moles.append(jnp.sum(og.reshape(NH, N, HID) * hm, axis=0))
    mole_all = _elu(jnp.concatenate(moles, axis=0))

    mole_b = _bf(mole_all)
    h2e = jnp.dot(mole_b, W2e_ref[...], preferred_element_type=jnp.float32)
    h2_b = _bf(h2e[:, :HID])
    es2 = h2e[:, HID:]
    ed2 = jax.lax.dot_general(
        W2d_ref[...], mole_b, (((1,), (1,)), ((), ())),
        preferred_element_type=jnp.float32)
    blocks = []
    for g in range(G):
        e = _leaky(es2[g * N:(g + 1) * N, :] + ed2[:, g * N:(g + 1) * N])
        blocks.append(jnp.exp2(e) * adjs[g])
    att2 = _att_blocks(blocks)
    aggs = [jnp.dot(att2[g * N:(g + 1) * N, :], h2_b[g * N:(g + 1) * N, :],
                    preferred_element_type=jnp.float32)
            for g in range(G)]
    mole_all = _elu(jnp.concatenate(aggs, axis=0)) + mole_all

    mole_b = _bf(mole_all)
    h3e = jnp.dot(mole_b, W3e_ref[...], preferred_element_type=jnp.float32)
    h3_b = _bf(h3e[:, :DOUT])
    es3 = h3e[:, DOUT:]
    ed3 = jax.lax.dot_general(
        W3d_ref[...], mole_b, (((1,), (1,)), ((), ())),
        preferred_element_type=jnp.float32)
    blocks = []
    for g in range(G):
        e = _leaky(es3[g * N:(g + 1) * N, :] + ed3[:, g * N:(g + 1) * N])
        blocks.append(jnp.exp2(e) * adjs[g])
    att3 = _att_blocks(blocks)
    outs = [jnp.dot(att3[g * N:(g + 1) * N, :], h3_b[g * N:(g + 1) * N, :],
                    preferred_element_type=jnp.float32)
            for g in range(G)]
    out_all = jnp.concatenate(outs, axis=0)

    res = jnp.dot(FCW_ref[...], _bf(out_all),
                  preferred_element_type=jnp.float32)
    res = res + fcb_ref[0, 0]
    for g in range(G):
        o_ref[g] = res[g:g + 1, :]


def kernel(atom_feature, weight, adj, W1, a1s, a1d, W2, a2s, a2d, W3, a3s, a3d,
           fc_w, fc_b):
    del weight

    LOG2E = 1.4426950408889634
    f32 = jnp.float32
    W1c = jnp.transpose(W1, (1, 0, 2)).reshape(DIN, HID)
    block_eye = jnp.repeat(jnp.eye(NH, dtype=f32), DH, axis=0)
    A1s = block_eye * (LOG2E * a1s.reshape(HID))[:, None]
    A1d = block_eye * (LOG2E * a1d.reshape(HID))[:, None]
    W1e = _bf(jnp.concatenate([W1c, W1c @ A1s], axis=1))
    W1d = _bf((W1c @ A1d).T)
    HM = block_eye.T
    W2e = _bf(jnp.concatenate([W2, W2 @ (LOG2E * a2s)[:, None]], axis=1))
    W2d = _bf((W2 @ (LOG2E * a2d)[:, None]).T)
    W3e = _bf(jnp.concatenate([W3, W3 @ (LOG2E * a3s)[:, None]], axis=1))
    W3d = _bf((W3 @ (LOG2E * a3d)[:, None]).T)
    FCW = _bf(jnp.kron(jnp.eye(G, dtype=f32), fc_w[None, :]))
    fcb = fc_b.reshape(1, 1)

    const = lambda b: (0, 0)
    grid = (B // G,)
    out = pl.pallas_call(
        _gat_kernel,
        grid=grid,
        in_specs=[
            pl.BlockSpec((G, N, DIN), lambda b: (b, 0, 0)),
            pl.BlockSpec((G, N, N), lambda b: (b, 0, 0)),
            pl.BlockSpec((DIN, HID + NH), const),
            pl.BlockSpec((NH, DIN), const),
            pl.BlockSpec((NH, HID), const),
            pl.BlockSpec((HID, HID + 1), const),
            pl.BlockSpec((1, HID), const),
            pl.BlockSpec((HID, DOUT + 1), const),
            pl.BlockSpec((1, HID), const),
            pl.BlockSpec((G, G * N), const),
            pl.BlockSpec((1, 1), const),
        ],
        out_specs=pl.BlockSpec((G, 1, DOUT), lambda b: (b, 0, 0)),
        out_shape=jax.ShapeDtypeStruct((B, 1, DOUT), jnp.float32),
        compiler_params=pltpu.CompilerParams(
            dimension_semantics=("parallel",)),
    )(atom_feature, adj, W1e, W1d, HM, W2e, W2d, W3e, W3d, FCW, fcb)
    return out.reshape(B, DOUT)


# speedup vs baseline: 1.0314x; 1.0314x over previous
import jax
import jax.numpy as jnp
from jax.experimental import pallas as pl
from jax.experimental.pallas import tpu as pltpu

B, N, DIN, NH, DH = 256, 128, 128, 8, 16
HID = NH * DH
DOUT = 512

G = 16


def _leaky(x):
    return jnp.maximum(x, 0.2 * x)


def _bf(x):
    return x.astype(jnp.bfloat16)


def _elu(x):
    return jnp.where(x > 0, x, jnp.exp(jnp.minimum(x, 0.0)) - 1.0)


def _norm_rows(p):
    return p / jnp.sum(p, axis=-1, keepdims=True)


def _gat_kernel(x_ref, adj_ref, W1c_ref, A1s_ref, A1d_ref, HM_ref,
                W2_ref, a2s_ref, a2d_ref, W3_ref, a3s_ref, a3d_ref,
                FCW_ref, fcb_ref, o_ref):
    adjs = [adj_ref[g] for g in range(G)]

    x_all = x_ref[...].reshape(G * N, DIN)
    h1_all = jnp.dot(_bf(x_all), _bf(W1c_ref[...]),
                     preferred_element_type=jnp.float32)
    es_all = jnp.dot(h1_all, A1s_ref[...], preferred_element_type=jnp.float32)
    ed_all = jax.lax.dot_general(
        A1d_ref[...], h1_all, (((0,), (1,)), ((), ())),
        preferred_element_type=jnp.float32)

    blocks = []
    for g in range(G):
        for i in range(NH):
            e = _leaky(es_all[g * N:(g + 1) * N, i:i + 1]
                       + ed_all[i:i + 1, g * N:(g + 1) * N])
            blocks.append(jnp.exp(e) * adjs[g])
    att1 = _norm_rows(jnp.concatenate(blocks, axis=0))

    hm = HM_ref[...][:, None, :]
    att1 = _bf(att1)
    h1_b = _bf(h1_all)
    moles = []
    for g in range(G):
        og = jnp.dot(att1[g * NH * N:(g + 1) * NH * N, :],
                     h1_b[g * N:(g + 1) * N, :],
                     preferred_element_type=jnp.float32)
        moles.append(jnp.sum(og.reshape(NH, N, HID) * hm, axis=0))
    mole_all = _elu(jnp.concatenate(moles, axis=0))

    h2_all = jnp.dot(_bf(mole_all), _bf(W2_ref[...]),
                     preferred_element_type=jnp.float32)
    es2 = jnp.dot(h2_all, a2s_ref[...], preferred_element_type=jnp.float32)
    ed2 = jax.lax.dot_general(
        a2d_ref[...], h2_all, (((0,), (1,)), ((), ())),
        preferred_element_type=jnp.float32)
    blocks = []
    for g in range(G):
        e = _leaky(es2[g * N:(g + 1) * N, :] + ed2[:, g * N:(g + 1) * N])
        blocks.append(jnp.exp(e) * adjs[g])
    att2 = _bf(_norm_rows(jnp.concatenate(blocks, axis=0)))
    h2_b = _bf(h2_all)
    aggs = [jnp.dot(att2[g * N:(g + 1) * N, :], h2_b[g * N:(g + 1) * N, :],
                    preferred_element_type=jnp.float32)
            for g in range(G)]
    mole_all = _elu(jnp.concatenate(aggs, axis=0)) + mole_all

    h3_all = jnp.dot(_bf(mole_all), _bf(W3_ref[...]),
                     preferred_element_type=jnp.float32)
    es3 = jnp.dot(h3_all, a3s_ref[...], preferred_element_type=jnp.float32)
    ed3 = jax.lax.dot_general(
        a3d_ref[...], h3_all, (((0,), (1,)), ((), ())),
        preferred_element_type=jnp.float32)
    blocks = []
    for g in range(G):
        e = _leaky(es3[g * N:(g + 1) * N, :] + ed3[:, g * N:(g + 1) * N])
        blocks.append(jnp.exp(e) * adjs[g])
    att3 = _bf(_norm_rows(jnp.concatenate(blocks, axis=0)))
    h3_b = _bf(h3_all)
    outs = [jnp.dot(att3[g * N:(g + 1) * N, :], h3_b[g * N:(g + 1) * N, :],
                    preferred_element_type=jnp.float32)
            for g in range(G)]
    out_all = jnp.concatenate(outs, axis=0)

    res = jnp.dot(FCW_ref[...], out_all, preferred_element_type=jnp.float32)
    res = res + fcb_ref[0, 0]
    for g in range(G):
        o_ref[g] = res[g:g + 1, :]


def kernel(atom_feature, weight, adj, W1, a1s, a1d, W2, a2s, a2d, W3, a3s, a3d,
           fc_w, fc_b):
    del weight

    W1c = jnp.transpose(W1, (1, 0, 2)).reshape(DIN, HID)
    block_eye = jnp.repeat(jnp.eye(NH, dtype=jnp.float32), DH, axis=0)
    A1s = block_eye * a1s.reshape(HID)[:, None]
    A1d = block_eye * a1d.reshape(HID)[:, None]
    HM = block_eye.T
    a2s_c = a2s[:, None]
    a2d_c = a2d[:, None]
    a3s_c = a3s[:, None]
    a3d_c = a3d[:, None]
    FCW = jnp.kron(jnp.eye(G, dtype=jnp.float32), fc_w[None, :])
    fcb = fc_b.reshape(1, 1)

    const = lambda b: (0, 0)
    grid = (B // G,)
    out = pl.pallas_call(
        _gat_kernel,
        grid=grid,
        in_specs=[
            pl.BlockSpec((G, N, DIN), lambda b: (b, 0, 0)),
            pl.BlockSpec((G, N, N), lambda b: (b, 0, 0)),
            pl.BlockSpec((DIN, HID), const),
            pl.BlockSpec((HID, NH), const),
            pl.BlockSpec((HID, NH), const),
            pl.BlockSpec((NH, HID), const),
            pl.BlockSpec((HID, HID), const),
            pl.BlockSpec((HID, 1), const),
            pl.BlockSpec((HID, 1), const),
            pl.BlockSpec((HID, DOUT), const),
            pl.BlockSpec((DOUT, 1), const),
            pl.BlockSpec((DOUT, 1), const),
            pl.BlockSpec((G, G * N), const),
            pl.BlockSpec((1, 1), const),
        ],
        out_specs=pl.BlockSpec((G, 1, DOUT), lambda b: (b, 0, 0)),
        out_shape=jax.ShapeDtypeStruct((B, 1, DOUT), jnp.float32),
        compiler_params=pltpu.CompilerParams(
            dimension_semantics=("parallel",)),
    )(atom_feature, adj, W1c, A1s, A1d, HM, W2, a2s_c, a2d_c, W3, a3s_c,
      a3d_c, FCW, fcb)
    return out.reshape(B, DOUT)
